# baseline (speedup 1.0000x reference)
import functools

import jax
import jax.numpy as jnp
from jax import lax
from jax.experimental import pallas as pl
from jax.experimental.pallas import tpu as pltpu
from jax.experimental.pallas import tpu_sc as plsc

BATCH = 16384
VOCAB = 101
EMBED = 128
NC = 2
NS = 16
NW = NC * NS
B_PER_W = BATCH // NW
CHUNK = 128
NCHUNK = B_PER_W // CHUNK

_mesh = plsc.VectorSubcoreMesh(core_axis_name="c", subcore_axis_name="s")


@functools.partial(
    pl.kernel,
    mesh=_mesh,
    out_type=(
        jax.ShapeDtypeStruct((NW * NCHUNK, CHUNK, EMBED), jnp.float32),
        jax.ShapeDtypeStruct((NW, B_PER_W), jnp.float32),
    ),
    scratch_types=[
        pltpu.VMEM((NCHUNK, CHUNK), jnp.int32),
        pltpu.VMEM((B_PER_W, EMBED), jnp.float32),
        pltpu.VMEM((B_PER_W,), jnp.float32),
        pltpu.VMEM_SHARED((VOCAB, EMBED), jnp.float32),
        pltpu.SemaphoreType.DMA,
        pltpu.SemaphoreType.DMA,
        pltpu.SemaphoreType.DMA,
        pltpu.SemaphoreType.DMA,
        pltpu.SemaphoreType.DMA,
    ],
)
def _gather_kernel(idx_hbm, table_hbm, out_hbm, mask_hbm, idx_v, rows_v,
                   ones_v, table_sh, g0, g1, g2, g3, wsem):
    sid = lax.axis_index("s")
    wid = sid * NC + lax.axis_index("c")
    gsems = (g0, g1, g2, g3)
    @pl.when(sid == 0)
    def _():
        pltpu.sync_copy(table_hbm, table_sh)

    pltpu.sync_copy(idx_hbm.at[wid], idx_v)

    def _fill(i):
        ones_v[pl.ds(i * 16, 16)] = jnp.full((16,), 1.0, jnp.float32)
    pl.loop(0, B_PER_W // 16)(_fill)
    mask_copy = pltpu.async_copy(ones_v, mask_hbm.at[wid], wsem)

    plsc.subcore_barrier()
    copies = [
        pltpu.async_copy(
            table_sh.at[idx_v.at[j]],
            rows_v.at[pl.ds(j * CHUNK, CHUNK)],
            gsems[j],
        )
        for j in range(NCHUNK)
    ]
    writes = []
    for j in range(NCHUNK):
        copies[j].wait()
        writes.append(
            pltpu.async_copy(
                rows_v.at[pl.ds(j * CHUNK, CHUNK)],
                out_hbm.at[wid * NCHUNK + j],
                wsem,
            )
        )
    mask_copy.wait()
    for w in writes:
        w.wait()


def kernel(indices, table):
    idx = indices.astype(jnp.int32).reshape(NW, NCHUNK, CHUNK)
    out, mask = _gather_kernel(idx, table)
    int_embeds = out.reshape(BATCH, 1, EMBED)
    return (int_embeds, mask.reshape(BATCH, 1))


# speedup vs baseline: 1.0730x; 1.0730x over previous
import functools

import jax
import jax.numpy as jnp
from jax import lax
from jax.experimental import pallas as pl
from jax.experimental.pallas import tpu as pltpu
from jax.experimental.pallas import tpu_sc as plsc

BATCH = 16384
VOCAB = 101
EMBED = 128
NC = 2
NS = 16
NW = NC * NS
B_PER_W = BATCH // NW
CHUNK = 64
NCHUNK = B_PER_W // CHUNK

_mesh = plsc.VectorSubcoreMesh(core_axis_name="c", subcore_axis_name="s")


@functools.partial(
    pl.kernel,
    mesh=_mesh,
    out_type=jax.ShapeDtypeStruct((NW * NCHUNK, CHUNK, EMBED), jnp.float32),
    scratch_types=[
        pltpu.VMEM((NCHUNK, CHUNK), jnp.int32),
        pltpu.VMEM((B_PER_W, EMBED), jnp.float32),
        pltpu.VMEM_SHARED((VOCAB, EMBED), jnp.float32),
        pltpu.SemaphoreType.DMA,
        pltpu.SemaphoreType.DMA,
        pltpu.SemaphoreType.DMA,
        pltpu.SemaphoreType.DMA,
        pltpu.SemaphoreType.DMA,
        pltpu.SemaphoreType.DMA,
        pltpu.SemaphoreType.DMA,
        pltpu.SemaphoreType.DMA,
        pltpu.SemaphoreType.DMA,
        pltpu.SemaphoreType.DMA,
    ],
)
def _gather_kernel(idx_hbm, table_hbm, out_hbm, idx_v, rows_v, table_sh,
                   g0, g1, g2, g3, g4, g5, g6, g7, wsem, isem):
    sid = lax.axis_index("s")
    wid = sid * NC + lax.axis_index("c")
    gsems = (g0, g1, g2, g3, g4, g5, g6, g7)
    idx_copy = pltpu.async_copy(idx_hbm.at[wid], idx_v, isem)
    @pl.when(sid < 12)
    def _():
        start = pl.multiple_of(sid * 8, 8)
        pltpu.sync_copy(
            table_hbm.at[pl.ds(start, 8)], table_sh.at[pl.ds(start, 8)]
        )

    @pl.when(sid == 12)
    def _():
        pltpu.sync_copy(
            table_hbm.at[pl.ds(96, VOCAB - 96)],
            table_sh.at[pl.ds(96, VOCAB - 96)],
        )

    plsc.subcore_barrier()
    idx_copy.wait()
    copies = [
        pltpu.async_copy(
            table_sh.at[idx_v.at[j]],
            rows_v.at[pl.ds(j * CHUNK, CHUNK)],
            gsems[j],
        )
        for j in range(NCHUNK)
    ]
    writes = []
    for j in range(NCHUNK):
        copies[j].wait()
        writes.append(
            pltpu.async_copy(
                rows_v.at[pl.ds(j * CHUNK, CHUNK)],
                out_hbm.at[wid * NCHUNK + j],
                wsem,
            )
        )
    for w in writes:
        w.wait()


def kernel(indices, table):
    idx = indices.astype(jnp.int32).reshape(NW, NCHUNK, CHUNK)
    out = _gather_kernel(idx, table)
    int_embeds = out.reshape(BATCH, 1, EMBED)
    mask = jnp.ones((BATCH, 1), dtype=jnp.float32)
    return (int_embeds, mask)
